# baseline (speedup 1.0000x reference)
import functools

import jax
import jax.numpy as jnp
from jax import lax
from jax.experimental import pallas as pl
from jax.experimental.pallas import tpu as pltpu
from jax.experimental.pallas import tpu_sc as plsc

B = 16384
D = 128
H = 128
G = 4 * H
BLK = 2048
HB = B // 2
NBH = HB // BLK
MROWS = 128
MCOLS = B // MROWS


def _sigmoid(z):
    return 0.5 * jnp.tanh(z * 0.5) + 0.5


def _tc_body_a(x_r, h_r, c_r, mblk_r, m2d_r, wih_r, whh_r, b1_r, b2_r,
               hc_r, pos_r, validv_r):
    i = pl.program_id(0)

    @pl.when(i == 0)
    def _compute_pos():
        mf = m2d_r[...].astype(jnp.float32)
        row = lax.broadcasted_iota(jnp.int32, (MROWS, MCOLS), 0).astype(jnp.float32)
        col = lax.broadcasted_iota(jnp.int32, (MROWS, MCOLS), 1).astype(jnp.float32)
        lt = (row <= col).astype(jnp.float32)
        sl = (col < row).astype(jnp.float32)
        t = lax.dot_general(mf, lt, (((1,), (0,)), ((), ())),
                            preferred_element_type=jnp.float32)
        rowsum = t[:, MCOLS - 1:MCOLS]
        rowoff = lax.dot_general(sl, rowsum, (((1,), (0,)), ((), ())),
                                 preferred_element_type=jnp.float32)
        a = t + rowoff
        valid = jnp.sum(mf)
        validv_r[...] = jnp.zeros((8, MCOLS), jnp.float32) + valid
        bidx = row * float(MCOLS) + col
        posf = jnp.where(mf > 0.0, a - 1.0, valid + bidx - a)
        pos_r[...] = posf.astype(jnp.int32)

    _lstm_block(x_r, h_r, c_r, mblk_r, wih_r, whh_r, b1_r, b2_r, hc_r)


def _tc_body_b(x_r, h_r, c_r, mblk_r, wih_r, whh_r, b1_r, b2_r, hc_r):
    _lstm_block(x_r, h_r, c_r, mblk_r, wih_r, whh_r, b1_r, b2_r, hc_r)


def _lstm_block(x_r, h_r, c_r, mblk_r, wih_r, whh_r, b1_r, b2_r, hc_r):
    x = x_r[...]
    h = h_r[...]
    c = c_r[...]
    gates = (lax.dot_general(x, wih_r[...], (((1,), (1,)), ((), ())),
                             preferred_element_type=jnp.float32)
             + lax.dot_general(h, whh_r[...], (((1,), (1,)), ((), ())),
                               preferred_element_type=jnp.float32)
             + b1_r[...] + b2_r[...])
    ig = _sigmoid(gates[:, 0:H])
    fg = _sigmoid(gates[:, H:2 * H])
    gg = jnp.tanh(gates[:, 2 * H:3 * H])
    og = _sigmoid(gates[:, 3 * H:4 * H])
    c1 = fg * c + ig * gg
    h1 = og * jnp.tanh(c1)

    mt = jnp.transpose(mblk_r[...], (1, 0))
    nsub = BLK // 128
    for j in range(nsub):
        sl = slice(j * 128, (j + 1) * 128)
        m = mt[:, j:j + 1] > 0
        hmv = jnp.where(m, h1[sl, :], h[sl, :])
        cmv = jnp.where(m, c1[sl, :], c[sl, :])
        hb = lax.bitcast_convert_type(hmv.astype(jnp.bfloat16),
                                      jnp.uint16).astype(jnp.uint32)
        cb = lax.bitcast_convert_type(cmv.astype(jnp.bfloat16),
                                      jnp.uint16).astype(jnp.uint32)
        hc_r[sl, :] = (hb << 16) | cb


def _tc_half(half, x2, h2, c2, m2d, wih, whh, b1, b2):
    off = half * NBH
    full = lambda shape: pl.BlockSpec(shape, lambda i: (0, 0))
    gblk = pl.BlockSpec((BLK, 128), lambda i: (i + off, 0))
    hblk = pl.BlockSpec((BLK, 128), lambda i: (i, 0))
    mblk = pl.BlockSpec((BLK // 128, MCOLS), lambda i: (i + off, 0))
    if half == 0:
        return pl.pallas_call(
            _tc_body_a,
            grid=(NBH,),
            in_specs=[gblk, gblk, gblk, mblk, full((MROWS, MCOLS)),
                      full((G, D)), full((G, H)), full((1, G)), full((1, G))],
            out_specs=[hblk, full((MROWS, MCOLS)), full((8, MCOLS))],
            out_shape=[
                jax.ShapeDtypeStruct((HB, H), jnp.uint32),
                jax.ShapeDtypeStruct((MROWS, MCOLS), jnp.int32),
                jax.ShapeDtypeStruct((8, MCOLS), jnp.float32),
            ],
        )(x2, h2, c2, m2d, m2d, wih, whh, b1, b2)
    return pl.pallas_call(
        _tc_body_b,
        grid=(NBH,),
        in_specs=[gblk, gblk, gblk, mblk,
                  full((G, D)), full((G, H)), full((1, G)), full((1, G))],
        out_specs=hblk,
        out_shape=jax.ShapeDtypeStruct((HB, H), jnp.uint32),
    )(x2, h2, c2, m2d, wih, whh, b1, b2)


def _sc_scatter_half(half, hc, pos2d, validv, x2, xo_ref, h_ref, c_ref):
    mesh = plsc.VectorSubcoreMesh(core_axis_name="c", subcore_axis_name="s")
    nw = mesh.num_cores * mesh.num_subcores
    rpw = HB // nw
    chunks = rpw // 128
    prow_off = half * (HB // MCOLS)
    gbase0 = half * HB

    @functools.partial(
        pl.kernel,
        out_type=(),
        mesh=mesh,
        scratch_types=[
            pltpu.VMEM((chunks, 128), jnp.int32),
            pltpu.VMEM((chunks * 4, 32), jnp.int32),
            pltpu.VMEM((8, MCOLS), jnp.float32),
            pltpu.VMEM((chunks, 128, H), jnp.uint32),
            pltpu.VMEM((chunks, 128, H), jnp.float32),
            pltpu.VMEM((chunks, 128, H), jnp.float32),
            pltpu.VMEM((128, H), jnp.float32),
            pltpu.SemaphoreType.DMA,
            pltpu.SemaphoreType.DMA,
            pltpu.SemaphoreType.DMA,
        ],
    )
    def scatter(hc_hbm, pos_hbm, validv_hbm, x_hbm, xo_hbm, hout_hbm, cout_hbm,
                pos_v, pos_s, vv, inb, hb, cb, xb, sem_in, sem_out, sem_xo):
        wid = lax.axis_index("s") * mesh.num_cores + lax.axis_index("c")
        base = wid * rpw
        gbase = gbase0 + base
        pltpu.sync_copy(pos_hbm.at[pl.ds(prow_off + wid * chunks, chunks)],
                        pos_v)
        for j in range(chunks):
            for k in range(4):
                for g in range(2):
                    pos_s[j * 4 + k, pl.ds(g * 16, 16)] = (
                        pos_v[j, pl.ds(k * 32 + g * 16, 16)])
        pltpu.sync_copy(validv_hbm, vv)
        valid_f = vv[0, pl.ds(0, 16)][0]
        valid_i = valid_f.astype(jnp.int32)
        special = jnp.logical_or(valid_f == 0.0, valid_f == float(B))
        loads = [pltpu.async_copy(hc_hbm.at[pl.ds(base + j * 128, 128)],
                                  inb.at[j], sem_in)
                 for j in range(chunks)]
        outs = []
        xo_cp = None
        for j in range(chunks):
            loads[j].wait()
            if xo_cp is not None:
                xo_cp.wait()

            def blk_body(rb, _, j=j):
                av = pos_v[j, pl.ds(rb * 16, 16)] < valid_i
                xsel = jnp.where(av, 1.0, 0.0)
                for r16 in range(16):
                    r = rb * 16 + r16
                    s = xsel[r16]
                    for g in range(H // 16):
                        v = inb[j, r, pl.ds(g * 16, 16)]
                        hf = lax.bitcast_convert_type(
                            v & jnp.uint32(0xFFFF0000), jnp.float32)
                        hb[j, r, pl.ds(g * 16, 16)] = hf
                        cb[j, r, pl.ds(g * 16, 16)] = lax.bitcast_convert_type(
                            v << 16, jnp.float32)
                        xb[r, pl.ds(g * 16, 16)] = hf * s
                return 0

            for k in range(4):
                lax.fori_loop(k * 2, k * 2 + 2, blk_body, 0)
                outs.append(pltpu.async_copy(
                    hb.at[j, pl.ds(k * 32, 32)],
                    hout_hbm.at[pos_s.at[j * 4 + k]], sem_out))
                outs.append(pltpu.async_copy(
                    cb.at[j, pl.ds(k * 32, 32)],
                    cout_hbm.at[pos_s.at[j * 4 + k]], sem_out))
            xo_cp = pltpu.async_copy(
                xb, xo_hbm.at[pl.ds(gbase + j * 128, 128)], sem_xo)
        xo_cp.wait()
        for cp in outs:
            cp.wait()

        @pl.when(special)
        def _special_x():
            pltpu.sync_copy(x_hbm.at[pl.ds(gbase, rpw)],
                            xo_hbm.at[pl.ds(gbase, rpw)])

    scatter(hc, pos2d, validv, x2, xo_ref, h_ref, c_ref)


def kernel(x, mask, h0, c0, W_ih, W_hh, b_ih, b_hh):
    x2 = x.reshape(B, D)
    h2 = h0.reshape(B, H)
    c2 = c0.reshape(B, H)
    m2d = mask.reshape(MROWS, MCOLS)
    b1 = b_ih.reshape(1, G)
    b2 = b_hh.reshape(1, G)

    hc_a, pos2d, validv = _tc_half(0, x2, h2, c2, m2d, W_ih, W_hh, b1, b2)
    hc_b = _tc_half(1, x2, h2, c2, m2d, W_ih, W_hh, b1, b2)

    xo_ref = jax.new_ref(pl.empty((B, H), jnp.float32))
    h_ref = jax.new_ref(pl.empty((B, H), jnp.float32))
    c_ref = jax.new_ref(pl.empty((B, H), jnp.float32))
    _sc_scatter_half(0, hc_a, pos2d, validv, x2, xo_ref, h_ref, c_ref)
    _sc_scatter_half(1, hc_b, pos2d, validv, x2, xo_ref, h_ref, c_ref)
    xo = jax.freeze(xo_ref)
    h_out = jax.freeze(h_ref)
    c_out = jax.freeze(c_ref)

    return (xo.reshape(B, 1, H),
            h_out.reshape(1, B, H),
            c_out.reshape(1, B, H))


# speedup vs baseline: 1.3343x; 1.3343x over previous
import functools

import jax
import jax.numpy as jnp
from jax import lax
from jax.experimental import pallas as pl
from jax.experimental.pallas import tpu as pltpu
from jax.experimental.pallas import tpu_sc as plsc

B = 16384
D = 128
H = 128
G = 4 * H
BLK = 2048
HB = B // 2
NBH = HB // BLK
MROWS = 128
MCOLS = B // MROWS


def _sigmoid(z):
    return 0.5 * jnp.tanh(z * 0.5) + 0.5


def _tc_body_a(x_r, h_r, c_r, mblk_r, m2d_r, wih_r, whh_r, b1_r, b2_r, xo_al_r,
               xo_r, hc_r, pos_r, validv_r, valid_s):
    i = pl.program_id(0)

    @pl.when(i == 0)
    def _compute_pos():
        mf = m2d_r[...].astype(jnp.float32)
        row = lax.broadcasted_iota(jnp.int32, (MROWS, MCOLS), 0).astype(jnp.float32)
        col = lax.broadcasted_iota(jnp.int32, (MROWS, MCOLS), 1).astype(jnp.float32)
        lt = (row <= col).astype(jnp.float32)
        sl = (col < row).astype(jnp.float32)
        t = lax.dot_general(mf, lt, (((1,), (0,)), ((), ())),
                            preferred_element_type=jnp.float32)
        rowsum = t[:, MCOLS - 1:MCOLS]
        rowoff = lax.dot_general(sl, rowsum, (((1,), (0,)), ((), ())),
                                 preferred_element_type=jnp.float32)
        a = t + rowoff
        valid = jnp.sum(mf)
        valid_s[0] = valid
        validv_r[...] = jnp.reshape(valid, (1, 1))
        bidx = row * float(MCOLS) + col
        posf = jnp.where(mf > 0.0, a - 1.0, valid + bidx - a)
        pos_r[...] = posf.astype(jnp.int32)

    _lstm_block(x_r, h_r, c_r, mblk_r, wih_r, whh_r, b1_r, b2_r,
                xo_r, hc_r, valid_s[0])


def _tc_body_b(x_r, h_r, c_r, mblk_r, wih_r, whh_r, b1_r, b2_r, validv_r,
               xo_al_r, xo_r, hc_r):
    _lstm_block(x_r, h_r, c_r, mblk_r, wih_r, whh_r, b1_r, b2_r,
                xo_r, hc_r, validv_r[...])


def _lstm_block(x_r, h_r, c_r, mblk_r, wih_r, whh_r, b1_r, b2_r,
                xo_r, hc_r, valid):
    special = jnp.logical_or(valid == 0.0, valid == float(B))
    x = x_r[...]
    h = h_r[...]
    c = c_r[...]
    gates = (lax.dot_general(x, wih_r[...], (((1,), (1,)), ((), ())),
                             preferred_element_type=jnp.float32)
             + lax.dot_general(h, whh_r[...], (((1,), (1,)), ((), ())),
                               preferred_element_type=jnp.float32)
             + b1_r[...] + b2_r[...])
    ig = _sigmoid(gates[:, 0:H])
    fg = _sigmoid(gates[:, H:2 * H])
    gg = jnp.tanh(gates[:, 2 * H:3 * H])
    og = _sigmoid(gates[:, 3 * H:4 * H])
    c1 = fg * c + ig * gg
    h1 = og * jnp.tanh(c1)

    mt = jnp.transpose(mblk_r[...], (1, 0))
    nsub = BLK // 128
    for j in range(nsub):
        sl = slice(j * 128, (j + 1) * 128)
        m = mt[:, j:j + 1] > 0
        xo_r[sl, :] = jnp.where(special, x[sl, :],
                                jnp.where(m, h1[sl, :], 0.0))
        hmv = jnp.where(m, h1[sl, :], h[sl, :])
        cmv = jnp.where(m, c1[sl, :], c[sl, :])
        hbv = lax.bitcast_convert_type(hmv.astype(jnp.bfloat16),
                                       jnp.uint16).astype(jnp.uint32)
        cbv = lax.bitcast_convert_type(cmv.astype(jnp.bfloat16),
                                       jnp.uint16).astype(jnp.uint32)
        hc_r[sl, :] = (hbv << 16) | cbv


def _tc_half(half, x2, h2, c2, m2d, wih, whh, b1, b2, xo_in, validv=None):
    off = half * NBH
    full = lambda shape: pl.BlockSpec(shape, lambda i: (0, 0))
    gblk = pl.BlockSpec((BLK, 128), lambda i: (i + off, 0))
    hblk = pl.BlockSpec((BLK, 128), lambda i: (i, 0))
    mblk = pl.BlockSpec((BLK // 128, MCOLS), lambda i: (i + off, 0))
    half_out = [
        jax.ShapeDtypeStruct((B, H), jnp.float32),
        jax.ShapeDtypeStruct((HB, H), jnp.uint32),
    ]
    anyspec = pl.BlockSpec(memory_space=pl.ANY)
    if half == 0:
        return pl.pallas_call(
            _tc_body_a,
            grid=(NBH,),
            in_specs=[gblk, gblk, gblk, mblk, full((MROWS, MCOLS)),
                      full((G, D)), full((G, H)), full((1, G)), full((1, G)),
                      anyspec],
            out_specs=[gblk, hblk, full((MROWS, MCOLS)), full((1, 1))],
            out_shape=half_out + [
                jax.ShapeDtypeStruct((MROWS, MCOLS), jnp.int32),
                jax.ShapeDtypeStruct((1, 1), jnp.float32),
            ],
            input_output_aliases={9: 0},
            scratch_shapes=[pltpu.SMEM((1,), jnp.float32)],
        )(x2, h2, c2, m2d, m2d, wih, whh, b1, b2, xo_in)
    return pl.pallas_call(
        _tc_body_b,
        grid=(NBH,),
        in_specs=[gblk, gblk, gblk, mblk,
                  full((G, D)), full((G, H)), full((1, G)), full((1, G)),
                  full((1, 1)), anyspec],
        out_specs=[gblk, hblk],
        out_shape=half_out,
        input_output_aliases={9: 0},
    )(x2, h2, c2, m2d, wih, whh, b1, b2, validv, xo_in)


def _sc_scatter_half(half, hc, pos2d, h_ref, c_ref):
    mesh = plsc.VectorSubcoreMesh(core_axis_name="c", subcore_axis_name="s")
    nw = mesh.num_cores * mesh.num_subcores
    rpw = HB // nw
    chunks = rpw // 128
    prow_off = half * (HB // MCOLS)

    @functools.partial(
        pl.kernel,
        out_type=(),
        mesh=mesh,
        scratch_types=[
            pltpu.VMEM((chunks, 128), jnp.int32),
            pltpu.VMEM((chunks, 128, H), jnp.uint32),
            pltpu.VMEM((chunks, 128, H), jnp.float32),
            pltpu.VMEM((chunks, 128, H), jnp.float32),
            pltpu.SemaphoreType.DMA,
            pltpu.SemaphoreType.DMA,
        ],
    )
    def scatter(hc_hbm, pos_hbm, hout_hbm, cout_hbm,
                pos_v, inb, hb, cb, sem_in, sem_out):
        wid = lax.axis_index("s") * mesh.num_cores + lax.axis_index("c")
        base = wid * rpw
        pltpu.sync_copy(pos_hbm.at[pl.ds(prow_off + wid * chunks, chunks)],
                        pos_v)
        loads = [pltpu.async_copy(hc_hbm.at[pl.ds(base + j * 128, 128)],
                                  inb.at[j], sem_in)
                 for j in range(chunks)]
        outs = []
        for j in range(chunks):
            loads[j].wait()

            def row_body(r, _, j=j):
                for g in range(H // 16):
                    v = inb[j, r, pl.ds(g * 16, 16)]
                    hb[j, r, pl.ds(g * 16, 16)] = lax.bitcast_convert_type(
                        v & jnp.uint32(0xFFFF0000), jnp.float32)
                    cb[j, r, pl.ds(g * 16, 16)] = lax.bitcast_convert_type(
                        v << 16, jnp.float32)
                return 0

            lax.fori_loop(0, 128, row_body, 0)
            outs.append(pltpu.async_copy(hb.at[j], hout_hbm.at[pos_v.at[j]],
                                         sem_out))
            outs.append(pltpu.async_copy(cb.at[j], cout_hbm.at[pos_v.at[j]],
                                         sem_out))
        for cp in outs:
            cp.wait()

    scatter(hc, pos2d, h_ref, c_ref)


def kernel(x, mask, h0, c0, W_ih, W_hh, b_ih, b_hh):
    x2 = x.reshape(B, D)
    h2 = h0.reshape(B, H)
    c2 = c0.reshape(B, H)
    m2d = mask.reshape(MROWS, MCOLS)
    b1 = b_ih.reshape(1, G)
    b2 = b_hh.reshape(1, G)

    xo0 = pl.empty((B, H), jnp.float32)
    xo1, hc_a, pos2d, validv = _tc_half(0, x2, h2, c2, m2d, W_ih, W_hh,
                                        b1, b2, xo0)
    xo, hc_b = _tc_half(1, x2, h2, c2, m2d, W_ih, W_hh, b1, b2, xo1, validv)

    h_ref = jax.new_ref(pl.empty((B, H), jnp.float32))
    c_ref = jax.new_ref(pl.empty((B, H), jnp.float32))
    _sc_scatter_half(0, hc_a, pos2d, h_ref, c_ref)
    _sc_scatter_half(1, hc_b, pos2d, h_ref, c_ref)
    h_out = jax.freeze(h_ref)
    c_out = jax.freeze(c_ref)

    return (xo.reshape(B, 1, H),
            h_out.reshape(1, B, H),
            c_out.reshape(1, B, H))
